# baseline (speedup 1.0000x reference)
import functools

import jax
import jax.numpy as jnp
from jax import lax
from jax.experimental import pallas as pl
from jax.experimental.pallas import tpu as pltpu
from jax.experimental.pallas import tpu_sc as plsc

_C = 0.1
_D = 64
_L = 16
_NC = 2
_NS = 16
_NW = _NC * _NS
_CH = 400


def _rsqrt16(x):
    bits = plsc.bitcast(x, jnp.int32)
    seed = jnp.int32(0x5F3759DF) - lax.shift_right_logical(bits, jnp.int32(1))
    y = plsc.bitcast(seed, jnp.float32)
    half_x = x * 0.5
    y = y * (1.5 - half_x * y * y)
    y = y * (1.5 - half_x * y * y)
    return y


def _sc_body(ids_hbm, idsT_hbm, table_hbm, s16_hbm, c16_hbm, out_hbm,
             idx_a, idx_b, gbuf_a, gbuf_b, obuf_a, obuf_b,
             s_v, c_v, pbuf, gs_a, gs_b, ws_a, ws_b):
    wid = lax.axis_index("s") * _NC + lax.axis_index("c")
    n_b = ids_hbm.shape[0] // _NW
    seq = ids_hbm.shape[1]
    rows_per_b = _CH // seq
    n_chunks = n_b // rows_per_b
    n_groups = _CH // _L
    b_base = wid * n_b

    pltpu.sync_copy(s16_hbm, s_v)
    pltpu.sync_copy(c16_hbm, c_v)
    sv = s_v[...]
    cv = c_v[...]
    sig = 1.0 / (1.0 + jnp.exp(-cv))
    mul = sv * sig
    coef = _C * sv * sv

    idxs = (idx_a, idx_b)
    gbufs = (gbuf_a, gbuf_b)
    obufs = (obuf_a, obuf_b)
    gsems = (gs_a, gs_b)
    wsems = (ws_a, ws_b)

    def fire(g, b):
        brow = b_base + g * rows_per_b
        for r in range(rows_per_b):
            pltpu.sync_copy(ids_hbm.at[brow + r],
                            idxs[b].at[pl.ds(r * seq, seq)])
        pltpu.async_copy(table_hbm.at[idxs[b]], gbufs[b], gsems[b])

    def wait_gather(b):
        pltpu.make_async_copy(table_hbm.at[idxs[b]], gbufs[b], gsems[b]).wait()

    def put(g, b):
        brow = b_base + g * rows_per_b
        for r in range(rows_per_b):
            pltpu.async_copy(obufs[b].at[pl.ds(r * seq, seq)],
                             out_hbm.at[brow + r], wsems[b])

    def wait_put(b):
        for r in range(rows_per_b):
            pltpu.make_async_copy(
                obufs[b].at[pl.ds(r * seq, seq)],
                out_hbm.at[b_base + r], wsems[b]).wait()

    def compute(src, dst, pbuf):
        @plsc.parallel_loop(0, n_groups, unroll=1)
        def group(j):
            for k in range(_L):
                i = j * _L + k
                p = jnp.zeros((_L,), jnp.float32)
                for q in range(4):
                    v = src[i, pl.ds(q * _L, _L)]
                    p = p + v * v
                pbuf[j, k, pl.ds(0, _L)] = p
            lane = lax.iota(jnp.int32, 16)
            jv = jnp.full((_L,), 0, jnp.int32) + j
            ns = jnp.zeros((_L,), jnp.float32)
            for c in range(_L):
                cvec = jnp.full((_L,), c, jnp.int32)
                ns = ns + plsc.load_gather(pbuf, [jv, lane, cvec])
            t = 1.0 + coef * ns
            r = _rsqrt16(t)
            fac = mul / (1.0 + t * r)
            for k in range(_L):
                i = j * _L + k
                f = jnp.broadcast_to(fac[k], (_L,))
                for q in range(4):
                    dst[i, pl.ds(q * _L, _L)] = src[i, pl.ds(q * _L, _L)] * f

    fire(0, 0)

    def pair_body(i, carry):
        g0 = 2 * i
        g1 = g0 + 1
        fire(g1, 1)

        @pl.when(i >= 1)
        def _():
            wait_put(0)
        wait_gather(0)
        compute(gbufs[0], obufs[0], pbuf)
        put(g0, 0)

        @pl.when(g1 + 1 < n_chunks)
        def _():
            fire(g1 + 1, 0)

        @pl.when(i >= 1)
        def _():
            wait_put(1)
        wait_gather(1)
        compute(gbufs[1], obufs[1], pbuf)
        put(g1, 1)
        return carry

    lax.fori_loop(0, n_chunks // 2, pair_body, 0)
    wait_put(0)
    wait_put(1)


def kernel(input_ids, embed_table, scale, curvature_param):
    ids = input_ids.astype(jnp.int32)
    s16 = jnp.broadcast_to(scale.astype(jnp.float32), (_L,))
    c16 = jnp.broadcast_to(curvature_param.astype(jnp.float32), (_L,))

    sc_call = functools.partial(
        pl.kernel,
        out_type=jax.ShapeDtypeStruct(
            (input_ids.shape[0], input_ids.shape[1], _D), jnp.float32),
        mesh=plsc.VectorSubcoreMesh(core_axis_name="c", subcore_axis_name="s"),
        compiler_params=pltpu.CompilerParams(
            needs_layout_passes=False, use_tc_tiling_on_sc=False),
        scratch_types=[
            pltpu.VMEM((_CH,), jnp.int32),
            pltpu.VMEM((_CH,), jnp.int32),
            pltpu.VMEM((_CH, _D), jnp.float32),
            pltpu.VMEM((_CH, _D), jnp.float32),
            pltpu.VMEM((_CH, _D), jnp.float32),
            pltpu.VMEM((_CH, _D), jnp.float32),
            pltpu.VMEM((_L,), jnp.float32),
            pltpu.VMEM((_L,), jnp.float32),
            pltpu.VMEM((_CH // _L, _L, 17), jnp.float32),
            pltpu.SemaphoreType.DMA,
            pltpu.SemaphoreType.DMA,
            pltpu.SemaphoreType.DMA,
            pltpu.SemaphoreType.DMA,
        ],
    )(_sc_body)
    return sc_call(ids, input_ids.T.astype(jnp.int32), embed_table, s16, c16)


# speedup vs baseline: 1.0378x; 1.0378x over previous
import functools

import jax
import jax.numpy as jnp
from jax import lax
from jax.experimental import pallas as pl
from jax.experimental.pallas import tpu as pltpu
from jax.experimental.pallas import tpu_sc as plsc

_C = 0.1
_D = 64
_L = 16
_NC = 2
_NS = 16
_NW = _NC * _NS
_CH = 400


def _rsqrt16(x):
    bits = plsc.bitcast(x, jnp.int32)
    seed = jnp.int32(0x5F3759DF) - lax.shift_right_logical(bits, jnp.int32(1))
    y = plsc.bitcast(seed, jnp.float32)
    half_x = x * 0.5
    y = y * (1.5 - half_x * y * y)
    y = y * (1.5 - half_x * y * y)
    return y


def _sc_body(idsT_hbm, table_hbm, s16_hbm, c16_hbm, out_hbm,
             idx_a, idx_b, gbuf_a, gbuf_b, obuf_a, obuf_b,
             s_v, c_v, pbuf, ids_blk, gs_a, gs_b, ws_a, ws_b):
    wid = lax.axis_index("s") * _NC + lax.axis_index("c")
    n_b = out_hbm.shape[0] // _NW
    seq = idsT_hbm.shape[0]
    rows_per_b = _CH // seq
    n_chunks = n_b // rows_per_b
    n_groups = _CH // _L
    b_base = wid * n_b
    half = n_b // 2

    pltpu.sync_copy(s16_hbm, s_v)
    pltpu.sync_copy(c16_hbm, c_v)
    sv = s_v[...]
    cv = c_v[...]
    sig = 1.0 / (1.0 + jnp.exp(-cv))
    mul = sv * sig
    coef = _C * sv * sv

    idxs = (idx_a, idx_b)
    gbufs = (gbuf_a, gbuf_b)
    obufs = (obuf_a, obuf_b)
    gsems = (gs_a, gs_b)
    wsems = (ws_a, ws_b)

    pltpu.sync_copy(idsT_hbm.at[:, pl.ds(b_base, half)], ids_blk)

    lane16 = lax.iota(jnp.int32, 16)
    t_offs = [16 * m for m in range(seq // _L)] + (
        [seq - _L] if seq % _L else [])

    def fire(g, b):
        refresh = g * rows_per_b - half

        @pl.when(refresh == 0)
        def _():
            pltpu.sync_copy(idsT_hbm.at[:, pl.ds(b_base + half, half)],
                            ids_blk)
        col0 = g * rows_per_b - jnp.where(g * rows_per_b >= half, half, 0)
        for r in range(rows_per_b):
            cvec = jnp.full((_L,), 0, jnp.int32) + (col0 + r)
            for off in t_offs:
                tv = lane16 + off
                vals = plsc.load_gather(ids_blk, [tv, cvec])
                idxs[b][pl.ds(r * seq + off, _L)] = vals
        pltpu.async_copy(table_hbm.at[idxs[b]], gbufs[b], gsems[b])

    def wait_gather(b):
        pltpu.make_async_copy(table_hbm.at[idxs[b]], gbufs[b], gsems[b]).wait()

    def put(g, b):
        brow = b_base + g * rows_per_b
        for r in range(rows_per_b):
            pltpu.async_copy(obufs[b].at[pl.ds(r * seq, seq)],
                             out_hbm.at[brow + r], wsems[b])

    def wait_put(b):
        for r in range(rows_per_b):
            pltpu.make_async_copy(
                obufs[b].at[pl.ds(r * seq, seq)],
                out_hbm.at[b_base + r], wsems[b]).wait()

    def compute(src, dst, pbuf):
        @plsc.parallel_loop(0, n_groups, unroll=1)
        def group(j):
            for k in range(_L):
                i = j * _L + k
                p = jnp.zeros((_L,), jnp.float32)
                for q in range(4):
                    v = src[i, pl.ds(q * _L, _L)]
                    p = p + v * v
                pbuf[j, k, pl.ds(0, _L)] = p
            lane = lax.iota(jnp.int32, 16)
            jv = jnp.full((_L,), 0, jnp.int32) + j
            ns = jnp.zeros((_L,), jnp.float32)
            for c in range(_L):
                cvec = jnp.full((_L,), c, jnp.int32)
                ns = ns + plsc.load_gather(pbuf, [jv, lane, cvec])
            t = 1.0 + coef * ns
            r = _rsqrt16(t)
            fac = mul / (1.0 + t * r)
            for k in range(_L):
                i = j * _L + k
                f = jnp.broadcast_to(fac[k], (_L,))
                for q in range(4):
                    dst[i, pl.ds(q * _L, _L)] = src[i, pl.ds(q * _L, _L)] * f

    fire(0, 0)

    def pair_body(i, carry):
        g0 = 2 * i
        g1 = g0 + 1
        fire(g1, 1)

        @pl.when(i >= 1)
        def _():
            wait_put(0)
        wait_gather(0)
        compute(gbufs[0], obufs[0], pbuf)
        put(g0, 0)

        @pl.when(g1 + 1 < n_chunks)
        def _():
            fire(g1 + 1, 0)

        @pl.when(i >= 1)
        def _():
            wait_put(1)
        wait_gather(1)
        compute(gbufs[1], obufs[1], pbuf)
        put(g1, 1)
        return carry

    lax.fori_loop(0, n_chunks // 2, pair_body, 0)
    wait_put(0)
    wait_put(1)


def kernel(input_ids, embed_table, scale, curvature_param):
    ids_t = input_ids.T.astype(jnp.int32)
    s16 = jnp.broadcast_to(scale.astype(jnp.float32), (_L,))
    c16 = jnp.broadcast_to(curvature_param.astype(jnp.float32), (_L,))

    sc_call = functools.partial(
        pl.kernel,
        out_type=jax.ShapeDtypeStruct(
            (input_ids.shape[0], input_ids.shape[1], _D), jnp.float32),
        mesh=plsc.VectorSubcoreMesh(core_axis_name="c", subcore_axis_name="s"),
        compiler_params=pltpu.CompilerParams(
            needs_layout_passes=False, use_tc_tiling_on_sc=False),
        scratch_types=[
            pltpu.VMEM((_CH,), jnp.int32),
            pltpu.VMEM((_CH,), jnp.int32),
            pltpu.VMEM((_CH, _D), jnp.float32),
            pltpu.VMEM((_CH, _D), jnp.float32),
            pltpu.VMEM((_CH, _D), jnp.float32),
            pltpu.VMEM((_CH, _D), jnp.float32),
            pltpu.VMEM((_L,), jnp.float32),
            pltpu.VMEM((_L,), jnp.float32),
            pltpu.VMEM((_CH // _L, _L, 17), jnp.float32),
            pltpu.VMEM((input_ids.shape[1],
                        input_ids.shape[0] // _NW // 2), jnp.int32),
            pltpu.SemaphoreType.DMA,
            pltpu.SemaphoreType.DMA,
            pltpu.SemaphoreType.DMA,
            pltpu.SemaphoreType.DMA,
        ],
    )(_sc_body)
    return sc_call(ids_t, embed_table, s16, c16)
